# baseline (speedup 1.0000x reference)
import functools

import jax
import jax.numpy as jnp
from jax import lax
from jax.experimental import pallas as pl
from jax.experimental.pallas import tpu as pltpu
from jax.experimental.pallas import tpu_sc as plsc

NUM_EMB = 100000
DIM = 32
BATCH = 16384
TIN = 200
NW = 32
BB = 128
NBLK = BATCH // BB
NBW = NBLK // NW
NG = 4
NO = 2
TG = 8
NGRP = TIN // TG

_TROWS = NUM_EMB * DIM // 128
_TBLK = 1000


def _sin_body(t_ref, o_ref):
    x = t_ref[...]
    col = lax.broadcasted_iota(jnp.int32, x.shape, 1)
    o_ref[...] = jnp.where(col % DIM == 0, x, jnp.sin(x))


def _sin_transform(table):
    flat = table.reshape(_TROWS, 128)
    out = pl.pallas_call(
        _sin_body,
        out_shape=jax.ShapeDtypeStruct((_TROWS, 128), jnp.float32),
        grid=(_TROWS // _TBLK,),
        in_specs=[pl.BlockSpec((_TBLK, 128), lambda i: (i, 0))],
        out_specs=pl.BlockSpec((_TBLK, 128), lambda i: (i, 0)),
    )(flat)
    return out.reshape(NUM_EMB, DIM)


_MESH = plsc.VectorSubcoreMesh(core_axis_name="c", subcore_axis_name="s")


@functools.partial(
    pl.kernel,
    mesh=_MESH,
    out_type=jax.ShapeDtypeStruct((TIN, DIM // 8, NBLK, 8, BB), jnp.float32),
    compiler_params=pltpu.CompilerParams(
        use_tc_tiling_on_sc=False, needs_layout_passes=False),
    scratch_types=[
        pltpu.VMEM((TIN, BB), jnp.int32),
        pltpu.VMEM((NG, BB, DIM), jnp.float32),
        pltpu.VMEM((NO, TG, DIM * BB), jnp.float32),
        pltpu.SemaphoreType.DMA((NG,)),
        pltpu.SemaphoreType.DMA((NO,)),
    ],
)
def _sc_gather(table_hbm, xt_hbm, out_hbm, idxT, rows_v, stg, g_sem, o_sem):
    wid = lax.axis_index("s") * 2 + lax.axis_index("c")
    lane = jnp.arange(16, dtype=jnp.int32)
    base0 = lane * BB
    base1 = base0 + 16 * BB

    def g_copy(t, p):
        return pltpu.make_async_copy(
            table_hbm.at[idxT.at[t]], rows_v.at[p], g_sem.at[p])

    def o_copies(t0, q, bt):
        return [
            pltpu.make_async_copy(
                stg.at[q, :, pl.ds(d8 * BB, BB)],
                out_hbm.at[pl.ds(t0, TG), d8 // 8, bt, d8 % 8, :],
                o_sem.at[q])
            for d8 in range(DIM)
        ]

    def process_block(blk, bt):
        b0 = bt * BB
        pltpu.sync_copy(xt_hbm.at[:, pl.ds(b0, BB)], idxT)

        for p in range(NG):
            g_copy(p, p).start()

        @pl.loop(0, TIN, step=TG)
        def _(tt):
            q = (tt // TG) % NO

            @pl.when(tt >= NO * TG)
            def _():
                for c in o_copies(tt, q, bt):
                    c.wait()

            for k in range(TG):
                t = tt + k
                p = k % NG
                g_copy(t, p).wait()

                tgt = stg.at[q, k]

                @pl.loop(0, BB, step=8)
                def _(bb0):
                    for u in range(8):
                        b = bb0 + u
                        v0 = rows_v[p, b, pl.ds(0, 16)]
                        v1 = rows_v[p, b, pl.ds(16, 16)]
                        plsc.store_scatter(tgt, [base0 + b], v0)
                        plsc.store_scatter(tgt, [base1 + b], v1)

                @pl.when(t + NG < TIN)
                def _():
                    g_copy(t + NG, p).start()

            for c in o_copies(tt, q, bt):
                c.start()

        for g in (NGRP - 2, NGRP - 1):
            for c in o_copies(g * TG, g % NO, bt):
                c.wait()

    @pl.loop(0, NBW)
    def _(blk):
        process_block(blk, wid * NBW + blk)


def kernel(x, table):
    table_t = _sin_transform(table)
    xt = jnp.transpose(x.astype(jnp.int32))
    out5 = _sc_gather(table_t, xt)
    y = jnp.transpose(out5, (2, 4, 0, 1, 3))
    return y.reshape(BATCH, TIN, DIM)


# speedup vs baseline: 1.0489x; 1.0489x over previous
import functools

import jax
import jax.numpy as jnp
from jax import lax
from jax.experimental import pallas as pl
from jax.experimental.pallas import tpu as pltpu
from jax.experimental.pallas import tpu_sc as plsc

NUM_EMB = 100000
DIM = 32
BATCH = 16384
TIN = 200
NW = 32
BB = 128
NBLK = BATCH // BB
NBW = NBLK // NW
NG = 4
NO = 2
TG = 8
NGRP = TIN // TG

_TROWS = NUM_EMB * DIM // 128
_TBLK = 1000


def _sin_body(t_ref, o_ref):
    x = t_ref[...]
    col = lax.broadcasted_iota(jnp.int32, x.shape, 1)
    o_ref[...] = jnp.where(col % DIM == 0, x, jnp.sin(x))


def _sin_transform(table):
    flat = table.reshape(_TROWS, 128)
    out = pl.pallas_call(
        _sin_body,
        out_shape=jax.ShapeDtypeStruct((_TROWS, 128), jnp.float32),
        grid=(_TROWS // _TBLK,),
        in_specs=[pl.BlockSpec((_TBLK, 128), lambda i: (i, 0))],
        out_specs=pl.BlockSpec((_TBLK, 128), lambda i: (i, 0)),
    )(flat)
    return out.reshape(NUM_EMB, DIM)


_MESH = plsc.VectorSubcoreMesh(core_axis_name="c", subcore_axis_name="s")


@functools.partial(
    pl.kernel,
    mesh=_MESH,
    out_type=jax.ShapeDtypeStruct((TIN, DIM // 8, NBLK, 8 * BB), jnp.float32),
    compiler_params=pltpu.CompilerParams(
        use_tc_tiling_on_sc=False, needs_layout_passes=False),
    scratch_types=[
        pltpu.VMEM((TIN, BB), jnp.int32),
        pltpu.VMEM((NG, BB, DIM), jnp.float32),
        pltpu.VMEM((NO, TG, DIM * BB), jnp.float32),
        pltpu.SemaphoreType.DMA((NG,)),
        pltpu.SemaphoreType.DMA((NO,)),
    ],
)
def _sc_gather(table_hbm, xt_hbm, out_hbm, idxT, rows_v, stg, g_sem, o_sem):
    wid = lax.axis_index("s") * 2 + lax.axis_index("c")
    lane = jnp.arange(16, dtype=jnp.int32)
    base0 = lane * BB
    base1 = base0 + 16 * BB

    def g_copy(t, p):
        return pltpu.make_async_copy(
            table_hbm.at[idxT.at[t]], rows_v.at[p], g_sem.at[p])

    def o_copies(t0, q, bt):
        return [
            pltpu.make_async_copy(
                stg.at[q, :, pl.ds(g * 8 * BB, 8 * BB)],
                out_hbm.at[pl.ds(t0, TG), g, bt],
                o_sem.at[q])
            for g in range(DIM // 8)
        ]

    def process_block(blk, bt):
        b0 = bt * BB
        pltpu.sync_copy(xt_hbm.at[:, pl.ds(b0, BB)], idxT)

        for p in range(NG):
            g_copy(p, p).start()

        @pl.loop(0, TIN, step=TG)
        def _(tt):
            q = (tt // TG) % NO

            @pl.when(tt >= NO * TG)
            def _():
                for c in o_copies(tt, q, bt):
                    c.wait()

            for k in range(TG):
                t = tt + k
                p = k % NG
                g_copy(t, p).wait()

                tgt = stg.at[q, k]

                @pl.loop(0, BB, step=8)
                def _(bb0):
                    for u in range(8):
                        b = bb0 + u
                        v0 = rows_v[p, b, pl.ds(0, 16)]
                        v1 = rows_v[p, b, pl.ds(16, 16)]
                        plsc.store_scatter(tgt, [base0 + b], v0)
                        plsc.store_scatter(tgt, [base1 + b], v1)

                @pl.when(t + NG < TIN)
                def _():
                    g_copy(t + NG, p).start()

            for c in o_copies(tt, q, bt):
                c.start()

        for g in (NGRP - 2, NGRP - 1):
            for c in o_copies(g * TG, g % NO, bt):
                c.wait()

    @pl.loop(0, NBW)
    def _(blk):
        process_block(blk, wid * NBW + blk)


def kernel(x, table):
    table_t = _sin_transform(table)
    xt = jnp.transpose(x.astype(jnp.int32))
    out4 = _sc_gather(table_t, xt)
    out5 = out4.reshape(TIN, DIM // 8, NBLK, 8, BB)
    y = jnp.transpose(out5, (2, 4, 0, 1, 3))
    return y.reshape(BATCH, TIN, DIM)
